# baseline (speedup 1.0000x reference)
import functools

import jax
import jax.numpy as jnp
from jax import lax
from jax.experimental import pallas as pl
from jax.experimental.pallas import tpu as pltpu
from jax.experimental.pallas import tpu_sc as plsc

NC, NS, L = 1, 16, 16
NW = NC * NS
B, N, S = 64, 128, 2
HW = 128 * 128
BPW = B // NW
PW = BPW * N
NCHUNK = PW // L


def _sc_loss(out_flat, ind_flat, mask_flat, target_2d):
    mesh = plsc.VectorSubcoreMesh(
        core_axis_name="c", subcore_axis_name="s",
        num_cores=NC, num_subcores=NS)

    @functools.partial(
        pl.kernel,
        out_type=jax.ShapeDtypeStruct((1,), jnp.float32),
        mesh=mesh,
        scratch_types=[
            pltpu.VMEM((BPW, N), jnp.int32),
            pltpu.VMEM((BPW, N), jnp.float32),
            pltpu.VMEM((PW,), jnp.float32),
            pltpu.VMEM((PW,), jnp.float32),
            pltpu.VMEM((PW,), jnp.int32),
            pltpu.VMEM((PW,), jnp.int32),
            pltpu.VMEM((PW,), jnp.int32),
            pltpu.VMEM((PW,), jnp.int32),
            pltpu.VMEM((PW,), jnp.float32),
            pltpu.VMEM((PW,), jnp.float32),
            pltpu.VMEM((2 * L,), jnp.float32),
            pltpu.VMEM((NW, 2 * L), jnp.float32),
            pltpu.VMEM_SHARED((NW, 2 * L), jnp.float32),
            pltpu.VMEM((L,), jnp.float32),
            pltpu.SemaphoreType.DMA,
        ],
    )
    def k(out_hbm, ind_hbm, mask_hbm, tgt_hbm, o_hbm,
          ind_v, m_v, t0_v, t1_v, idx0_v, idx1_v, tix0_v, tix1_v, p0_v, p1_v,
          part_v, all_v, shared, res_v, sem):
        wid = lax.axis_index("s") * NC + lax.axis_index("c")
        b0 = wid * BPW
        p0 = wid * PW
        iota = lax.iota(jnp.int32, L)
        c0 = pltpu.async_copy(ind_hbm.at[pl.ds(b0, BPW)], ind_v, sem)
        c1 = pltpu.async_copy(mask_hbm.at[pl.ds(b0, BPW)], m_v, sem)
        for i in range(NCHUNK):
            sl = pl.ds(i * L, L)
            e = (p0 + i * L + iota) * S
            tix0_v[sl] = e
            tix1_v[sl] = e + 1
        gs = []
        for j in range(BPW):
            gs.append(pltpu.async_copy(
                tgt_hbm.at[tix0_v.at[pl.ds(j * N, N)]],
                t0_v.at[pl.ds(j * N, N)], sem))
            gs.append(pltpu.async_copy(
                tgt_hbm.at[tix1_v.at[pl.ds(j * N, N)]],
                t1_v.at[pl.ds(j * N, N)], sem))
        c0.wait()
        for j in range(BPW):
            base = (b0 + j) * (S * HW)
            for i in range(N // L):
                c = ind_v[j, pl.ds(i * L, L)]
                idx0_v[pl.ds(j * N + i * L, L)] = c + base
                idx1_v[pl.ds(j * N + i * L, L)] = c + (base + HW)
        for j in range(BPW):
            gs.append(pltpu.async_copy(
                out_hbm.at[idx0_v.at[pl.ds(j * N, N)]],
                p0_v.at[pl.ds(j * N, N)], sem))
            gs.append(pltpu.async_copy(
                out_hbm.at[idx1_v.at[pl.ds(j * N, N)]],
                p1_v.at[pl.ds(j * N, N)], sem))
        c1.wait()
        for g in gs:
            g.wait()
        lacc = jnp.zeros((L,), jnp.float32)
        macc = jnp.zeros((L,), jnp.float32)
        for i in range(NCHUNK):
            sl = pl.ds(i * L, L)
            m = m_v[i // (N // L), pl.ds((i % (N // L)) * L, L)]
            lacc = (lacc + jnp.abs(p0_v[sl] * m - t0_v[sl] * m)
                    + jnp.abs(p1_v[sl] * m - t1_v[sl] * m))
            macc = macc + m
        part_v[pl.ds(0, L)] = lacc
        part_v[pl.ds(L, L)] = macc
        pltpu.sync_copy(part_v, shared.at[wid])
        plsc.subcore_barrier()

        @pl.when(wid == 0)
        def _():
            pltpu.sync_copy(shared, all_v)
            lt = jnp.zeros((L,), jnp.float32)
            mt = jnp.zeros((L,), jnp.float32)
            for w in range(NW):
                lt = lt + all_v[w, pl.ds(0, L)]
                mt = mt + all_v[w, pl.ds(L, L)]
            num = jnp.float32(0.0)
            den = jnp.float32(0.0001)
            for i in range(L):
                num = num + lt[i]
                den = den + mt[i]
            res_v[...] = (jnp.full((L,), num, jnp.float32)
                          / jnp.full((L,), den, jnp.float32))
            pltpu.sync_copy(res_v.at[pl.ds(0, 1)], o_hbm)

    return k(out_flat, ind_flat, mask_flat, target_2d)


def kernel(out, target, ind, mask):
    res = _sc_loss(out.reshape(-1), ind, mask, target.reshape(-1))
    return res.reshape(())
